# baseline (speedup 1.0000x reference)
import functools

import jax
import jax.numpy as jnp
from jax import lax
from jax.experimental import pallas as pl
from jax.experimental.pallas import tpu as pltpu
from jax.experimental.pallas import tpu_sc as plsc

D = 128
H = 8
TW = 2 * H
PW = TW // 2
NC, NS = 2, 16
NW = NC * NS
L = 16


def _splat_i32(v):
    return jnp.full((L,), v, dtype=jnp.int32)


def _project_body(x_ref, w1_ref, b1_ref, t_ref):
    x = x_ref[...]
    dn = (((1,), (0,)), ((), ()))
    po = 2.0 * lax.dot_general(x, w1_ref[0:D, :], dn,
                               preferred_element_type=jnp.float32)
    pi = 2.0 * (lax.dot_general(x, w1_ref[D:2 * D, :], dn,
                                preferred_element_type=jnp.float32)
                + b1_ref[...])
    def pack(half):
        lo = lax.convert_element_type(
            lax.bitcast_convert_type(
                half[:, 0:H // 2].astype(jnp.bfloat16), jnp.uint16),
            jnp.uint32)
        hi = lax.convert_element_type(
            lax.bitcast_convert_type(
                half[:, H // 2:H].astype(jnp.bfloat16), jnp.uint16),
            jnp.uint32)
        return lax.bitcast_convert_type(lo | (hi << 16), jnp.int32)

    t_ref[...] = jnp.concatenate([pack(po), pack(pi)], axis=1)


def _make_sc_kernel(n: int, epw: int):
    mesh = plsc.VectorSubcoreMesh(core_axis_name="c", subcore_axis_name="s")
    epad = NW * epw

    @functools.partial(
        pl.kernel,
        mesh=mesh,
        out_type=jax.ShapeDtypeStruct((epad,), jnp.float32),
        compiler_params=pltpu.CompilerParams(
            needs_layout_passes=False, use_tc_tiling_on_sc=False),
        scratch_types=[
            pltpu.VMEM((n, PW), jnp.int32),
            pltpu.VMEM((epw,), jnp.int32),
            pltpu.VMEM((epw,), jnp.int32),
            pltpu.VMEM((epw,), jnp.float32),
            pltpu.VMEM((L, L), jnp.float32),
            pltpu.SemaphoreType.DMA,
            pltpu.SemaphoreType.DMA,
            pltpu.SemaphoreType.DMA,
            pltpu.SemaphoreType.DMA,
        ],
    )
    def sc_edge_mlp(tblh, ro, ri, wb, out, tbl, idx_o, idx_i, outb, wbv,
                    s0, s1, s2, s3):
        wid = lax.axis_index("s") * NC + lax.axis_index("c")
        base = pl.multiple_of(wid * epw, 8)
        ct = pltpu.async_copy(tblh, tbl, s0)
        co = pltpu.async_copy(ro.at[pl.ds(base, epw)], idx_o, s1)
        ci = pltpu.async_copy(ri.at[pl.ds(base, epw)], idx_i, s2)
        cw = pltpu.async_copy(wb, wbv, s3)
        cw.wait()
        co.wait()
        ci.wait()
        ct.wait()

        w2s = [wbv[h, :] for h in range(H)]
        csum = wbv[H, :]
        ms = []
        for h in range(H):
            csum = csum + w2s[h]
            ms.append(-2.0 * w2s[h])

        sh16 = _splat_i32(16)
        himask = _splat_i32(-65536)

        def unpack2(word):
            lo = plsc.bitcast(lax.shift_left(word, sh16), jnp.float32)
            hi = plsc.bitcast(lax.bitwise_and(word, himask), jnp.float32)
            return lo, hi

        @plsc.parallel_loop(0, epw // L, 1, unroll=8)
        def block(k):
            off = k * L
            eo = idx_o[pl.ds(off, L)]
            ei = idx_i[pl.ds(off, L)]
            svals = [None] * H
            for w in range(H // 2):
                wo = plsc.load_gather(tbl, [eo, _splat_i32(w)])
                wi = plsc.load_gather(tbl, [ei, _splat_i32(w + H // 2)])
                olo, ohi = unpack2(wo)
                ilo, ihi = unpack2(wi)
                svals[w] = olo + ilo
                svals[w + H // 2] = ohi + ihi
            acc = csum
            for h in range(H):
                e2 = jnp.exp(svals[h])
                acc = acc + ms[h] / (e2 + 1.0)
            outb[pl.ds(off, L)] = 1.0 / (1.0 + jnp.exp(-acc))

        pltpu.sync_copy(outb, out.at[pl.ds(base, epw)])

    return sc_edge_mlp


def kernel(X, Ri, Ro, W1, b1, W2, b2):
    Bb, N, Dd = X.shape
    E = Ri.shape[1]

    x = X.reshape(N, Dd)
    b1r = b1.reshape(1, H)

    packed = pl.pallas_call(
        _project_body,
        out_shape=jax.ShapeDtypeStruct((N, PW), jnp.int32),
    )(x, W1, b1r)

    wb = jnp.concatenate(
        [W2.reshape(H), b2.reshape(1),
         jnp.zeros((L - H - 1,), jnp.float32)])
    wb = jnp.tile(wb[:, None], (1, L))

    epw = -(-E // NW)
    epw = -(-epw // L) * L
    epad = NW * epw
    ro = Ro.reshape(E).astype(jnp.int32)
    ri = Ri.reshape(E).astype(jnp.int32)
    if epad != E:
        ro = jnp.pad(ro, (0, epad - E))
        ri = jnp.pad(ri, (0, epad - E))

    y = _make_sc_kernel(N, epw)(packed, ro, ri, wb)

    if epad != E:
        y = y[:E]
    return y.reshape(Bb, E)


# speedup vs baseline: 1.0789x; 1.0789x over previous
import functools

import jax
import jax.numpy as jnp
from jax import lax
from jax.experimental import pallas as pl
from jax.experimental.pallas import tpu as pltpu
from jax.experimental.pallas import tpu_sc as plsc

D = 128
H = 8
TW = 2 * H
PW = TW // 2
NC, NS = 2, 16
NW = NC * NS
L = 16


def _splat_i32(v):
    return jnp.full((L,), v, dtype=jnp.int32)


def _project_body(x_ref, w1_ref, b1_ref, t_ref):
    x = x_ref[...]
    dn = (((1,), (0,)), ((), ()))
    po = 2.0 * lax.dot_general(x, w1_ref[0:D, :], dn,
                               preferred_element_type=jnp.float32)
    pi = 2.0 * (lax.dot_general(x, w1_ref[D:2 * D, :], dn,
                                preferred_element_type=jnp.float32)
                + b1_ref[...])
    def pack(half):
        lo = lax.convert_element_type(
            lax.bitcast_convert_type(
                half[:, 0:H // 2].astype(jnp.bfloat16), jnp.uint16),
            jnp.uint32)
        hi = lax.convert_element_type(
            lax.bitcast_convert_type(
                half[:, H // 2:H].astype(jnp.bfloat16), jnp.uint16),
            jnp.uint32)
        return lax.bitcast_convert_type(lo | (hi << 16), jnp.int32)

    t_ref[...] = jnp.concatenate([pack(po), pack(pi)], axis=1)


def _make_sc_kernel(n: int, epw: int):
    mesh = plsc.VectorSubcoreMesh(core_axis_name="c", subcore_axis_name="s")
    epad = NW * epw

    @functools.partial(
        pl.kernel,
        mesh=mesh,
        out_type=jax.ShapeDtypeStruct((epad,), jnp.float32),
        compiler_params=pltpu.CompilerParams(
            needs_layout_passes=False, use_tc_tiling_on_sc=False),
        scratch_types=[
            pltpu.VMEM((n, PW), jnp.int32),
            pltpu.VMEM((epw,), jnp.int32),
            pltpu.VMEM((epw,), jnp.int32),
            pltpu.VMEM((epw,), jnp.float32),
            pltpu.VMEM((L, L), jnp.float32),
            pltpu.SemaphoreType.DMA,
            pltpu.SemaphoreType.DMA,
            pltpu.SemaphoreType.DMA,
            pltpu.SemaphoreType.DMA,
        ],
    )
    def sc_edge_mlp(tblh, ro, ri, wb, out, tbl, idx_o, idx_i, outb, wbv,
                    s0, s1, s2, s3):
        wid = lax.axis_index("s") * NC + lax.axis_index("c")
        base = pl.multiple_of(wid * epw, 8)
        ct = pltpu.async_copy(tblh, tbl, s0)
        co = pltpu.async_copy(ro.at[pl.ds(base, epw)], idx_o, s1)
        ci = pltpu.async_copy(ri.at[pl.ds(base, epw)], idx_i, s2)
        cw = pltpu.async_copy(wb, wbv, s3)
        cw.wait()
        co.wait()
        ci.wait()
        ct.wait()

        w2s = [wbv[h, :] for h in range(H)]
        csum = wbv[H, :]
        ms = []
        for h in range(H):
            csum = csum + w2s[h]
            ms.append(-2.0 * w2s[h])

        sh16 = _splat_i32(16)
        himask = _splat_i32(-65536)

        def unpack2(word):
            lo = plsc.bitcast(lax.shift_left(word, sh16), jnp.float32)
            hi = plsc.bitcast(lax.bitwise_and(word, himask), jnp.float32)
            return lo, hi

        @plsc.parallel_loop(0, epw // L, 1, unroll=5)
        def block(k):
            off = k * L
            eo = idx_o[pl.ds(off, L)]
            ei = idx_i[pl.ds(off, L)]
            svals = [None] * H
            for w in range(H // 2):
                wo = plsc.load_gather(tbl, [eo, _splat_i32(w)])
                wi = plsc.load_gather(tbl, [ei, _splat_i32(w + H // 2)])
                olo, ohi = unpack2(wo)
                ilo, ihi = unpack2(wi)
                svals[w] = olo + ilo
                svals[w + H // 2] = ohi + ihi
            acc = csum
            for h in range(H):
                e2 = jnp.exp(svals[h])
                acc = acc + ms[h] / (e2 + 1.0)
            outb[pl.ds(off, L)] = 1.0 / (1.0 + jnp.exp(-acc))

        pltpu.sync_copy(outb, out.at[pl.ds(base, epw)])

    return sc_edge_mlp


def kernel(X, Ri, Ro, W1, b1, W2, b2):
    Bb, N, Dd = X.shape
    E = Ri.shape[1]

    x = X.reshape(N, Dd)
    b1r = b1.reshape(1, H)

    packed = pl.pallas_call(
        _project_body,
        out_shape=jax.ShapeDtypeStruct((N, PW), jnp.int32),
    )(x, W1, b1r)

    wb = jnp.concatenate(
        [W2.reshape(H), b2.reshape(1),
         jnp.zeros((L - H - 1,), jnp.float32)])
    wb = jnp.tile(wb[:, None], (1, L))

    epw = -(-E // NW)
    epw = -(-epw // L) * L
    epad = NW * epw
    ro = Ro.reshape(E).astype(jnp.int32)
    ri = Ri.reshape(E).astype(jnp.int32)
    if epad != E:
        ro = jnp.pad(ro, (0, epad - E))
        ri = jnp.pad(ri, (0, epad - E))

    y = _make_sc_kernel(N, epw)(packed, ro, ri, wb)

    if epad != E:
        y = y[:E]
    return y.reshape(Bb, E)
